# Initial kernel scaffold; baseline (speedup 1.0000x reference)
import jax
import jax.numpy as jnp
from jax.experimental import pallas as pl


def kernel(head_idx, rel_idx, tail_idx, qual_type_idx, qual_value_idx, qual_seg, entity_emb, relation_emb, W1, b1, W2, b2, Wa, ba, va, Wp, bp):
    raise NotImplementedError("write your pallas kernel here")


# speedup vs baseline: 3.7661x; 3.7661x over previous
import functools

import jax
import jax.numpy as jnp
from jax import lax
from jax.experimental import pallas as pl
from jax.experimental.pallas import tpu as pltpu
from jax.experimental.pallas import tpu_sc as plsc

D = 256
BQ = 256
BT = 256
DEN = 128
W_OH = BQ + 8

_NC, _NS = 2, 16
_NW = _NC * _NS
_CH = 128


def _sc_gather_all(entity_emb, relation_emb, idx_val, idx_ht, idx_typ, idx_rel):
    nv, nht, nt, nr = (idx_val.shape[0], idx_ht.shape[0],
                       idx_typ.shape[0], idx_rel.shape[0])
    mesh = plsc.VectorSubcoreMesh(core_axis_name="c", subcore_axis_name="s")

    @functools.partial(
        pl.kernel,
        mesh=mesh,
        out_type=(
            jax.ShapeDtypeStruct((nv, D), jnp.float32),
            jax.ShapeDtypeStruct((nht, D), jnp.float32),
            jax.ShapeDtypeStruct((nt, D), jnp.float32),
            jax.ShapeDtypeStruct((nr, D), jnp.float32),
        ),
        scratch_types=[
            pltpu.VMEM((_CH,), jnp.int32),
            pltpu.VMEM((_CH, D), jnp.float32),
            pltpu.SemaphoreType.DMA,
        ],
    )
    def k(ent, rel, iv, iht, it, ir, ov, oht, ot, orr, idx_v, rows_v, sem):
        wid = lax.axis_index("s") * _NC + lax.axis_index("c")

        def gather(idx_hbm, table, out_hbm, n_rows):
            per_w = n_rows // _NW
            base_w = wid * per_w

            def body(i, carry):
                base = base_w + i * _CH
                pltpu.sync_copy(idx_hbm.at[pl.ds(base, _CH)], idx_v)
                pltpu.async_copy(table.at[idx_v], rows_v, sem).wait()
                pltpu.sync_copy(rows_v, out_hbm.at[pl.ds(base, _CH)])
                return carry

            lax.fori_loop(0, per_w // _CH, body, 0)

        gather(iv, ent, ov, nv)
        gather(iht, ent, oht, nht)
        gather(it, rel, ot, nt)
        gather(ir, rel, orr, nr)

    return k(entity_emb, relation_emb, idx_val, idx_ht, idx_typ, idx_rel)


def _mlp_agg_body(s0_ref, typ_ref, val_ref, seg_ref, w1a_ref, w1b_ref, b1_ref,
                  w2_ref, b2_ref, wa_ref, ba_ref, va_ref, out_ref):
    i = pl.program_id(0)

    @pl.when(i == 0)
    def _():
        out_ref[...] = jnp.zeros_like(out_ref)

    dot = functools.partial(
        lax.dot_general, preferred_element_type=jnp.float32)
    mm = lambda a, b: dot(a, b, (((1,), (0,)), ((), ())))

    h = jnp.maximum(
        mm(typ_ref[...], w1a_ref[...]) + mm(val_ref[...], w1b_ref[...])
        + b1_ref[...], 0.0)
    pq = mm(h, w2_ref[...]) + b2_ref[...]
    va = va_ref[...]
    t = jnp.tanh(mm(pq, wa_ref[...]) + ba_ref[...])
    s = jnp.sum(t * va, axis=1, keepdims=True)
    c = jnp.sum(jnp.abs(va))
    e = jnp.exp(s - c)

    seg = seg_ref[0]
    s0 = s0_ref[i]
    base = pl.multiple_of((s0 // 8) * 8, 8)
    oh = (lax.broadcasted_iota(jnp.int32, (W_OH, BQ), 0) == (seg - base)
          ).astype(jnp.float32)
    x = jnp.concatenate(
        [e * pq, jnp.broadcast_to(e, (BQ, DEN))], axis=1)
    contrib = mm(oh, x)
    cur = out_ref[pl.ds(base, W_OH), :]
    out_ref[pl.ds(base, W_OH), :] = cur + contrib


def _proj_body(h_ref, r_ref, t_ref, nd_ref, wp1_ref, wp2_ref, wp3_ref,
               wp4_ref, bp_ref, out_ref):
    dot = functools.partial(
        lax.dot_general, preferred_element_type=jnp.float32)
    mm = lambda a, b: dot(a, b, (((1,), (0,)), ((), ())))
    nd = nd_ref[...]
    agg = nd[:, :D] / nd[:, D:D + 1]
    out_ref[...] = (mm(h_ref[...], wp1_ref[...]) + mm(r_ref[...], wp2_ref[...])
                    + mm(t_ref[...], wp3_ref[...]) + mm(agg, wp4_ref[...])
                    + bp_ref[...])


def kernel(head_idx, rel_idx, tail_idx, qual_type_idx, qual_value_idx,
           qual_seg, entity_emb, relation_emb, W1, b1, W2, b2, Wa, ba, va,
           Wp, bp):
    T = head_idx.shape[0]
    Q = qual_type_idx.shape[0]
    nbq = Q // BQ
    nbt = T // BT

    idx_ht = jnp.concatenate([head_idx, tail_idx]).astype(jnp.int32)
    val_e, ht_e, typ_e, rel_e = _sc_gather_all(
        entity_emb, relation_emb, qual_value_idx.astype(jnp.int32), idx_ht,
        qual_type_idx.astype(jnp.int32), rel_idx.astype(jnp.int32))

    seg = qual_seg.astype(jnp.int32)
    seg_starts = seg[::BQ]
    seg3d = seg.reshape(nbq, 1, BQ)

    full = lambda shape: pl.BlockSpec(shape, lambda i, *_: (0, 0))
    row = lambda shape: pl.BlockSpec(shape, lambda i, *_: (i, 0))
    numden = pl.pallas_call(
        _mlp_agg_body,
        grid_spec=pltpu.PrefetchScalarGridSpec(
            num_scalar_prefetch=1,
            grid=(nbq,),
            in_specs=[
                row((BQ, D)), row((BQ, D)),
                pl.BlockSpec((1, 1, BQ), lambda i, *_: (i, 0, 0)),
                full((D, D)), full((D, D)), full((1, D)),
                full((D, D)), full((1, D)),
                full((D, D)), full((1, D)), full((1, D)),
            ],
            out_specs=pl.BlockSpec((T + BQ, D + DEN), lambda i, *_: (0, 0)),
        ),
        out_shape=jax.ShapeDtypeStruct((T + BQ, D + DEN), jnp.float32),
    )(seg_starts, typ_e, val_e, seg3d, W1[:D], W1[D:], b1.reshape(1, D),
      W2, b2.reshape(1, D), Wa, ba.reshape(1, D), va.reshape(1, D))

    tokens = pl.pallas_call(
        _proj_body,
        grid=(nbt,),
        in_specs=[
            pl.BlockSpec((BT, D), lambda i: (i, 0)),
            pl.BlockSpec((BT, D), lambda i: (i, 0)),
            pl.BlockSpec((BT, D), lambda i, n=nbt: (i + n, 0)),
            pl.BlockSpec((BT, D + DEN), lambda i: (i, 0)),
            pl.BlockSpec((D, D), lambda i: (0, 0)),
            pl.BlockSpec((D, D), lambda i: (0, 0)),
            pl.BlockSpec((D, D), lambda i: (0, 0)),
            pl.BlockSpec((D, D), lambda i: (0, 0)),
            pl.BlockSpec((1, D), lambda i: (0, 0)),
        ],
        out_specs=pl.BlockSpec((BT, D), lambda i: (i, 0)),
        out_shape=jax.ShapeDtypeStruct((T, D), jnp.float32),
    )(ht_e, rel_e, ht_e, numden, Wp[:D], Wp[D:2 * D], Wp[2 * D:3 * D],
      Wp[3 * D:], bp.reshape(1, D))
    return tokens


# speedup vs baseline: 4.2088x; 1.1175x over previous
import functools

import jax
import jax.numpy as jnp
from jax import lax
from jax.experimental import pallas as pl
from jax.experimental.pallas import tpu as pltpu
from jax.experimental.pallas import tpu_sc as plsc

D = 256
BQ = 256
BT = 256
DEN = 128
W_OH = BQ + 8

_NC, _NS = 2, 16
_NW = _NC * _NS
_CH = 128


_NBUF = 3


def _sc_gather_pair(table_a, table_b, idx_a, idx_b):
    na, nb = idx_a.shape[0], idx_b.shape[0]
    pa, pb = na // _NW, nb // _NW
    mesh = plsc.VectorSubcoreMesh(core_axis_name="c", subcore_axis_name="s")

    @functools.partial(
        pl.kernel,
        mesh=mesh,
        out_type=(
            jax.ShapeDtypeStruct((na, D), jnp.float32),
            jax.ShapeDtypeStruct((nb, D), jnp.float32),
        ),
        scratch_types=[
            pltpu.VMEM((pa,), jnp.int32),
            pltpu.VMEM((pb,), jnp.int32),
            pltpu.VMEM((_NBUF, _CH, D), jnp.float32),
            pltpu.SemaphoreType.DMA((_NBUF,)),
            pltpu.SemaphoreType.DMA((_NBUF,)),
        ],
    )
    def k(ta, tb, ia, ib, oa, ob, ia_v, ib_v, rows_v, gsem, osem):
        wid = lax.axis_index("s") * _NC + lax.axis_index("c")
        pltpu.sync_copy(ia.at[pl.ds(wid * pa, pa)], ia_v)
        pltpu.sync_copy(ib.at[pl.ds(wid * pb, pb)], ib_v)

        work = [(ia_v, ta, oa, wid * pa, j) for j in range(pa // _CH)]
        work += [(ib_v, tb, ob, wid * pb, j) for j in range(pb // _CH)]
        n = len(work)
        gd, oc = [None] * n, [None] * n
        for i in range(n + 1):
            if i < n:
                slot = i % _NBUF
                if i >= _NBUF:
                    oc[i - _NBUF].wait()
                idx_v, table, _, _, j = work[i]
                gd[i] = pltpu.async_copy(
                    table.at[idx_v.at[pl.ds(j * _CH, _CH)]],
                    rows_v.at[slot], gsem.at[slot])
            if i >= 1:
                p = i - 1
                _, _, out_hbm, base_w, j = work[p]
                gd[p].wait()
                oc[p] = pltpu.async_copy(
                    rows_v.at[p % _NBUF],
                    out_hbm.at[pl.ds(base_w + j * _CH, _CH)],
                    osem.at[p % _NBUF])
        for p in range(max(n - _NBUF, 0), n):
            oc[p].wait()

    return k(table_a, table_b, idx_a, idx_b)


def _mlp_agg_body(s0_ref, typ_ref, val_ref, seg_ref, w1a_ref, w1b_ref, b1_ref,
                  w2_ref, b2_ref, wa_ref, ba_ref, va_ref, out_ref):
    i = pl.program_id(0)

    dot = functools.partial(
        lax.dot_general, preferred_element_type=jnp.float32)
    mm = lambda a, b: dot(a, b, (((1,), (0,)), ((), ())))

    h = jnp.maximum(
        mm(typ_ref[...], w1a_ref[...]) + mm(val_ref[...], w1b_ref[...])
        + b1_ref[...], 0.0)
    pq = mm(h, w2_ref[...]) + b2_ref[...]
    va = va_ref[...]
    t = jnp.tanh(mm(pq, wa_ref[...]) + ba_ref[...])
    s = jnp.sum(t * va, axis=1, keepdims=True)
    c = jnp.sum(jnp.abs(va))
    e = jnp.exp(s - c)

    seg = seg_ref[0]
    s0 = s0_ref[i]
    base = pl.multiple_of((s0 // 8) * 8, 8)
    oh = (lax.broadcasted_iota(jnp.int32, (W_OH, BQ), 0) == (seg - base)
          ).astype(jnp.float32)
    x = jnp.concatenate(
        [e * pq, jnp.broadcast_to(e, (BQ, DEN))], axis=1)
    contrib = mm(oh, x)
    prev_end = jnp.where(
        i == 0, 0, (s0_ref[jnp.maximum(i - 1, 0)] // 8) * 8 + W_OH)
    rows = base + lax.broadcasted_iota(jnp.int32, (W_OH, 1), 0)
    cur = out_ref[pl.ds(base, W_OH), :]
    cur = jnp.where(rows < prev_end, cur, 0.0)
    out_ref[pl.ds(base, W_OH), :] = cur + contrib


def _proj_body(h_ref, r_ref, t_ref, nd_ref, wp1_ref, wp2_ref, wp3_ref,
               wp4_ref, bp_ref, out_ref):
    dot = functools.partial(
        lax.dot_general, preferred_element_type=jnp.float32)
    mm = lambda a, b: dot(a, b, (((1,), (0,)), ((), ())))
    nd = nd_ref[...]
    agg = nd[:, :D] / nd[:, D:D + 1]
    out_ref[...] = (mm(h_ref[...], wp1_ref[...]) + mm(r_ref[...], wp2_ref[...])
                    + mm(t_ref[...], wp3_ref[...]) + mm(agg, wp4_ref[...])
                    + bp_ref[...])


def kernel(head_idx, rel_idx, tail_idx, qual_type_idx, qual_value_idx,
           qual_seg, entity_emb, relation_emb, W1, b1, W2, b2, Wa, ba, va,
           Wp, bp):
    T = head_idx.shape[0]
    Q = qual_type_idx.shape[0]
    nbq = Q // BQ
    nbt = T // BT

    idx_ht = jnp.concatenate([head_idx, tail_idx]).astype(jnp.int32)
    val_e, typ_e = _sc_gather_pair(
        entity_emb, relation_emb, qual_value_idx.astype(jnp.int32),
        qual_type_idx.astype(jnp.int32))
    ht_e, rel_e = _sc_gather_pair(
        entity_emb, relation_emb, idx_ht, rel_idx.astype(jnp.int32))

    seg = qual_seg.astype(jnp.int32)
    seg_starts = seg[::BQ]
    seg3d = seg.reshape(nbq, 1, BQ)

    full = lambda shape: pl.BlockSpec(shape, lambda i, *_: (0, 0))
    row = lambda shape: pl.BlockSpec(shape, lambda i, *_: (i, 0))
    numden = pl.pallas_call(
        _mlp_agg_body,
        grid_spec=pltpu.PrefetchScalarGridSpec(
            num_scalar_prefetch=1,
            grid=(nbq,),
            in_specs=[
                row((BQ, D)), row((BQ, D)),
                pl.BlockSpec((1, 1, BQ), lambda i, *_: (i, 0, 0)),
                full((D, D)), full((D, D)), full((1, D)),
                full((D, D)), full((1, D)),
                full((D, D)), full((1, D)), full((1, D)),
            ],
            out_specs=pl.BlockSpec((T + BQ, D + DEN), lambda i, *_: (0, 0)),
        ),
        out_shape=jax.ShapeDtypeStruct((T + BQ, D + DEN), jnp.float32),
    )(seg_starts, typ_e, val_e, seg3d, W1[:D], W1[D:], b1.reshape(1, D),
      W2, b2.reshape(1, D), Wa, ba.reshape(1, D), va.reshape(1, D))

    tokens = pl.pallas_call(
        _proj_body,
        grid=(nbt,),
        in_specs=[
            pl.BlockSpec((BT, D), lambda i: (i, 0)),
            pl.BlockSpec((BT, D), lambda i: (i, 0)),
            pl.BlockSpec((BT, D), lambda i, n=nbt: (i + n, 0)),
            pl.BlockSpec((BT, D + DEN), lambda i: (i, 0)),
            pl.BlockSpec((D, D), lambda i: (0, 0)),
            pl.BlockSpec((D, D), lambda i: (0, 0)),
            pl.BlockSpec((D, D), lambda i: (0, 0)),
            pl.BlockSpec((D, D), lambda i: (0, 0)),
            pl.BlockSpec((1, D), lambda i: (0, 0)),
        ],
        out_specs=pl.BlockSpec((BT, D), lambda i: (i, 0)),
        out_shape=jax.ShapeDtypeStruct((T, D), jnp.float32),
    )(ht_e, rel_e, ht_e, numden, Wp[:D], Wp[D:2 * D], Wp[2 * D:3 * D],
      Wp[3 * D:], bp.reshape(1, D))
    return tokens
